# baseline (speedup 1.0000x reference)
import functools

import jax
import jax.numpy as jnp
from jax import lax
from jax.experimental import pallas as pl
from jax.experimental.pallas import tpu as pltpu
from jax.experimental.pallas import tpu_sc as plsc

N = 10000
E = 160000
D = 256
BN_EPS = 1e-05

NC = 2
NS = 16
L = 16
NW = NC * NS

OWN = 320
ACC_ROWS = OWN + 8
SCAN_C = 1600
NCHUNK = E // SCAN_C
GB = 64
PEND = SCAN_C + 2 * GB


def _segsum_body(x_hbm, row_hbm, col_hbm, out_hbm,
                 rowb0, colb0, rowb1, colb1, pend_p,
                 gidx0a, gidx0b, gidx1a, gidx1b, grows0, grows1, acc,
                 semr, semc, sg0a, sg0b, sg1a, sg1b):
    c = lax.axis_index("c")
    s = lax.axis_index("s")
    g = c * NS + s
    base = g * OWN
    lane15 = jnp.full((L,), L - 1, jnp.int32)

    def zero_row(r, _):
        for j in range(D // L):
            acc[r, pl.ds(j * L, L)] = jnp.zeros((L,), jnp.float32)
        return ()
    lax.fori_loop(0, ACC_ROWS, zero_row, ())

    def start_stage(k, rb, cb):
        pltpu.async_copy(row_hbm.at[pl.ds(k * SCAN_C, SCAN_C)], rb, semr)
        pltpu.async_copy(col_hbm.at[pl.ds(k * SCAN_C, SCAN_C)], cb, semc)

    def wait_stage(k, rb, cb):
        pltpu.make_async_copy(row_hbm.at[pl.ds(k * SCAN_C, SCAN_C)],
                              rb, semr).wait()
        pltpu.make_async_copy(col_hbm.at[pl.ds(k * SCAN_C, SCAN_C)],
                              cb, semc).wait()

    HB = GB // 2

    def start_gather(b, slot):
        boff = b * GB
        gia, gib = (gidx0a, gidx0b) if slot == 0 else (gidx1a, gidx1b)
        gr = grows0 if slot == 0 else grows1
        sema, semb = (sg0a, sg0b) if slot == 0 else (sg1a, sg1b)
        for t in range(HB // L):
            gia[pl.ds(t * L, L)] = jnp.right_shift(
                pend_p[pl.ds(boff + t * L, L)], 9)
        for t in range(HB // L):
            gib[pl.ds(t * L, L)] = jnp.right_shift(
                pend_p[pl.ds(boff + HB + t * L, L)], 9)
        pltpu.async_copy(x_hbm.at[gia], gr.at[pl.ds(0, HB)], sema)
        pltpu.async_copy(x_hbm.at[gib], gr.at[pl.ds(HB, HB)], semb)

    def wait_gather(slot):
        gia, gib = (gidx0a, gidx0b) if slot == 0 else (gidx1a, gidx1b)
        gr = grows0 if slot == 0 else grows1
        sema, semb = (sg0a, sg0b) if slot == 0 else (sg1a, sg1b)
        pltpu.make_async_copy(x_hbm.at[gia], gr.at[pl.ds(0, HB)], sema).wait()
        pltpu.make_async_copy(x_hbm.at[gib], gr.at[pl.ds(HB, HB)], semb).wait()

    def add_batch(b, slot):
        boff = b * GB
        gbuf = grows0 if slot == 0 else grows1

        def grp(t, _):
            dv = pend_p[pl.ds(boff + t * L, L)] & 511
            for e in range(L):
                d = dv[e]
                gr = t * L + e
                for j in range(D // L):
                    plsc.addupdate(acc.at[d, pl.ds(j * L, L)],
                                   gbuf[gr, pl.ds(j * L, L)])
            return ()
        lax.fori_loop(0, GB // L, grp, ())

    def process_batches(nfull):
        @pl.when(nfull > 0)
        def _():
            start_gather(0, 0)

            def pair(p, _):
                b0 = p * 2

                @pl.when(b0 + 1 < nfull)
                def _():
                    start_gather(b0 + 1, 1)
                wait_gather(0)
                add_batch(b0, 0)

                @pl.when(b0 + 1 < nfull)
                def _():
                    @pl.when(b0 + 2 < nfull)
                    def _():
                        start_gather(b0 + 2, 0)
                    wait_gather(1)
                    add_batch(b0 + 1, 1)
                return ()
            lax.fori_loop(0, (nfull + 1) // 2, pair, ())

    def scan_chunk(k, np_, rb, cb):
        npv0 = jnp.full((L,), np_, jnp.int32)

        def scan_vec(i, npv):
            r = rb[pl.ds(i * L, L)]
            cv = cb[pl.ds(i * L, L)]
            u = r - base
            m = (u >= 0) & (u < OWN)
            pc = plsc.cumsum(m.astype(jnp.int32))
            pos = npv + pc - 1
            plsc.store_scatter(pend_p, [pos], cv * 512 + u, mask=m)
            bc = jnp.take_along_axis(pc, lane15, axis=0,
                                     mode="promise_in_bounds")
            return npv + bc
        npv = lax.fori_loop(0, SCAN_C // L, scan_vec, npv0)
        return npv[0]

    def do_chunk(k, np_, rb, cb, rb_next, cb_next):
        wait_stage(k, rb, cb)
        np_ = scan_chunk(k, np_, rb, cb)

        @pl.when(k + 1 < NCHUNK)
        def _():
            start_stage(k + 1, rb_next, cb_next)

        nfull = np_ // GB
        process_batches(nfull)
        rem = np_ - nfull * GB

        @pl.when(nfull > 0)
        def _():
            for t in range(GB // L):
                v = pend_p[pl.ds(nfull * GB + t * L, L)]
                pend_p[pl.ds(t * L, L)] = v
        return rem

    start_stage(0, rowb0, colb0)

    def pairchunk(p, np_):
        k0 = p * 2
        np_ = do_chunk(k0, np_, rowb0, colb0, rowb1, colb1)
        np_ = do_chunk(k0 + 1, np_, rowb1, colb1, rowb0, colb0)
        return np_
    np_ = lax.fori_loop(0, NCHUNK // 2, pairchunk, jnp.int32(0))

    for t in range(GB // L):
        pend_p[pl.ds(np_ + t * L, L)] = jnp.full((L,), OWN, jnp.int32)
    process_batches((np_ + GB - 1) // GB)

    @pl.when(g < NW - 1)
    def _():
        pltpu.sync_copy(acc.at[pl.ds(0, OWN)], out_hbm.at[pl.ds(base, OWN)])

    @pl.when(g == NW - 1)
    def _():
        pltpu.sync_copy(acc.at[pl.ds(0, N - (NW - 1) * OWN)],
                        out_hbm.at[pl.ds(base, N - (NW - 1) * OWN)])


_segsum = functools.partial(
    pl.kernel,
    out_type=jax.ShapeDtypeStruct((N, D), jnp.float32),
    mesh=plsc.VectorSubcoreMesh(
        core_axis_name="c", subcore_axis_name="s", num_cores=NC, num_subcores=NS
    ),
    scratch_types=[
        pltpu.VMEM((SCAN_C,), jnp.int32),
        pltpu.VMEM((SCAN_C,), jnp.int32),
        pltpu.VMEM((SCAN_C,), jnp.int32),
        pltpu.VMEM((SCAN_C,), jnp.int32),
        pltpu.VMEM((PEND,), jnp.int32),
        pltpu.VMEM((GB // 2,), jnp.int32),
        pltpu.VMEM((GB // 2,), jnp.int32),
        pltpu.VMEM((GB // 2,), jnp.int32),
        pltpu.VMEM((GB // 2,), jnp.int32),
        pltpu.VMEM((GB, D), jnp.float32),
        pltpu.VMEM((GB, D), jnp.float32),
        pltpu.VMEM((ACC_ROWS, D), jnp.float32),
        pltpu.SemaphoreType.DMA,
        pltpu.SemaphoreType.DMA,
        pltpu.SemaphoreType.DMA,
        pltpu.SemaphoreType.DMA,
        pltpu.SemaphoreType.DMA,
        pltpu.SemaphoreType.DMA,
    ],
    compiler_params=pltpu.CompilerParams(needs_layout_passes=False),
)(_segsum_body)


BLK = 1000
NBLK = N // BLK


def _mlp_body(x_ref, agg_ref, w1_ref, b1_ref, w2_ref, b2_ref,
              h_ref, s1_ref, s2_ref):
    i = pl.program_id(0)
    xa = x_ref[...] + agg_ref[...]
    h1 = jnp.maximum(
        jnp.dot(xa, w1_ref[...], preferred_element_type=jnp.float32) + b1_ref[...], 0.0)
    h = jnp.maximum(
        jnp.dot(h1, w2_ref[...], preferred_element_type=jnp.float32) + b2_ref[...], 0.0)
    h_ref[...] = h

    @pl.when(i == 0)
    def _():
        s1_ref[...] = jnp.zeros_like(s1_ref)
        s2_ref[...] = jnp.zeros_like(s2_ref)

    s1_ref[pl.ds(i, 1), :] = jnp.sum(h, axis=0, keepdims=True)
    s2_ref[pl.ds(i, 1), :] = jnp.sum(h * h, axis=0, keepdims=True)


def _mlp_stats(x, agg, W1, b1, W2, b2):
    return pl.pallas_call(
        _mlp_body,
        grid=(NBLK,),
        in_specs=[
            pl.BlockSpec((BLK, D), lambda i: (i, 0)),
            pl.BlockSpec((BLK, D), lambda i: (i, 0)),
            pl.BlockSpec((D, 2 * D), lambda i: (0, 0)),
            pl.BlockSpec((2 * D,), lambda i: (0,)),
            pl.BlockSpec((2 * D, D), lambda i: (0, 0)),
            pl.BlockSpec((D,), lambda i: (0,)),
        ],
        out_specs=[
            pl.BlockSpec((BLK, D), lambda i: (i, 0)),
            pl.BlockSpec((16, D), lambda i: (0, 0)),
            pl.BlockSpec((16, D), lambda i: (0, 0)),
        ],
        out_shape=[
            jax.ShapeDtypeStruct((N, D), jnp.float32),
            jax.ShapeDtypeStruct((16, D), jnp.float32),
            jax.ShapeDtypeStruct((16, D), jnp.float32),
        ],
    )(x, agg, W1, b1, W2, b2)


def _norm_body(h_ref, agg_ref, sc_ref, bi_ref, o_ref):
    m = (agg_ref[...] > 0).astype(jnp.float32)
    o_ref[...] = (h_ref[...] * sc_ref[0:1, :] + bi_ref[0:1, :]) * m


def _norm_mask(h, agg, scale, bias):
    return pl.pallas_call(
        _norm_body,
        grid=(NBLK,),
        in_specs=[
            pl.BlockSpec((BLK, D), lambda i: (i, 0)),
            pl.BlockSpec((BLK, D), lambda i: (i, 0)),
            pl.BlockSpec((8, D), lambda i: (0, 0)),
            pl.BlockSpec((8, D), lambda i: (0, 0)),
        ],
        out_specs=pl.BlockSpec((BLK, D), lambda i: (i, 0)),
        out_shape=jax.ShapeDtypeStruct((N, D), jnp.float32),
    )(h, agg, scale, bias)


def kernel(x, edge_index, W1, b1, W2, b2, gamma, beta):
    row = edge_index[0]
    col = edge_index[1]
    agg = _segsum(x, row, col)
    h, s1, s2 = _mlp_stats(x, agg, W1, b1, W2, b2)
    mean = jnp.sum(s1, axis=0) / N
    var = jnp.sum(s2, axis=0) / N - mean * mean
    rstd = 1.0 / jnp.sqrt(var + BN_EPS)
    inv_sqrt_n = 1.0 / jnp.sqrt(jnp.float32(N))
    scale = gamma * rstd * inv_sqrt_n
    bias = (beta - mean * gamma * rstd) * inv_sqrt_n
    scale_b = jnp.broadcast_to(scale[None, :], (8, D))
    bias_b = jnp.broadcast_to(bias[None, :], (8, D))
    return _norm_mask(h, agg, scale_b, bias_b)


# speedup vs baseline: 1.0408x; 1.0408x over previous
import functools

import jax
import jax.numpy as jnp
from jax import lax
from jax.experimental import pallas as pl
from jax.experimental.pallas import tpu as pltpu
from jax.experimental.pallas import tpu_sc as plsc

N = 10000
E = 160000
D = 256
BN_EPS = 1e-05

NC = 2
NS = 16
L = 16
NW = NC * NS

OWN = 320
ACC_ROWS = OWN + 8
SCAN_C = 1600
NCHUNK = E // SCAN_C
GB = 64
BMIN = 4
PEND = SCAN_C + (BMIN + 1) * GB


def _segsum_body(x_hbm, row_hbm, col_hbm, out_hbm,
                 rowb0, colb0, rowb1, colb1, pend_p,
                 gidx0a, gidx0b, gidx1a, gidx1b, grows0, grows1, acc,
                 semr, semc, sg0a, sg0b, sg1a, sg1b):
    c = lax.axis_index("c")
    s = lax.axis_index("s")
    g = c * NS + s
    base = g * OWN
    lane15 = jnp.full((L,), L - 1, jnp.int32)

    def zero_row(r, _):
        for j in range(D // L):
            acc[r, pl.ds(j * L, L)] = jnp.zeros((L,), jnp.float32)
        return ()
    lax.fori_loop(0, ACC_ROWS, zero_row, ())

    def start_stage(k, rb, cb):
        pltpu.async_copy(row_hbm.at[pl.ds(k * SCAN_C, SCAN_C)], rb, semr)
        pltpu.async_copy(col_hbm.at[pl.ds(k * SCAN_C, SCAN_C)], cb, semc)

    def wait_stage(k, rb, cb):
        pltpu.make_async_copy(row_hbm.at[pl.ds(k * SCAN_C, SCAN_C)],
                              rb, semr).wait()
        pltpu.make_async_copy(col_hbm.at[pl.ds(k * SCAN_C, SCAN_C)],
                              cb, semc).wait()

    HB = GB // 2

    def start_gather(b, slot):
        boff = b * GB
        gia, gib = (gidx0a, gidx0b) if slot == 0 else (gidx1a, gidx1b)
        gr = grows0 if slot == 0 else grows1
        sema, semb = (sg0a, sg0b) if slot == 0 else (sg1a, sg1b)
        for t in range(HB // L):
            gia[pl.ds(t * L, L)] = jnp.right_shift(
                pend_p[pl.ds(boff + t * L, L)], 9)
        for t in range(HB // L):
            gib[pl.ds(t * L, L)] = jnp.right_shift(
                pend_p[pl.ds(boff + HB + t * L, L)], 9)
        pltpu.async_copy(x_hbm.at[gia], gr.at[pl.ds(0, HB)], sema)
        pltpu.async_copy(x_hbm.at[gib], gr.at[pl.ds(HB, HB)], semb)

    def wait_gather(slot):
        gia, gib = (gidx0a, gidx0b) if slot == 0 else (gidx1a, gidx1b)
        gr = grows0 if slot == 0 else grows1
        sema, semb = (sg0a, sg0b) if slot == 0 else (sg1a, sg1b)
        pltpu.make_async_copy(x_hbm.at[gia], gr.at[pl.ds(0, HB)], sema).wait()
        pltpu.make_async_copy(x_hbm.at[gib], gr.at[pl.ds(HB, HB)], semb).wait()

    def add_batch(b, slot):
        boff = b * GB
        gbuf = grows0 if slot == 0 else grows1

        def grp(t, _):
            dv = pend_p[pl.ds(boff + t * L, L)] & 511
            for e in range(L):
                d = dv[e]
                gr = t * L + e
                for j in range(D // L):
                    plsc.addupdate(acc.at[d, pl.ds(j * L, L)],
                                   gbuf[gr, pl.ds(j * L, L)])
            return ()
        lax.fori_loop(0, GB // L, grp, ())

    def process_batches(nfull):
        @pl.when(nfull > 0)
        def _():
            start_gather(0, 0)

            def pair(p, _):
                b0 = p * 2

                @pl.when(b0 + 1 < nfull)
                def _():
                    start_gather(b0 + 1, 1)
                wait_gather(0)
                add_batch(b0, 0)

                @pl.when(b0 + 1 < nfull)
                def _():
                    @pl.when(b0 + 2 < nfull)
                    def _():
                        start_gather(b0 + 2, 0)
                    wait_gather(1)
                    add_batch(b0 + 1, 1)
                return ()
            lax.fori_loop(0, (nfull + 1) // 2, pair, ())

    def scan_chunk(k, np_, rb, cb):
        npv0 = jnp.full((L,), np_, jnp.int32)

        def scan_vec(i, npv):
            r = rb[pl.ds(i * L, L)]
            cv = cb[pl.ds(i * L, L)]
            u = r - base
            m = (u >= 0) & (u < OWN)
            pc = plsc.cumsum(m.astype(jnp.int32))
            pos = npv + pc - 1
            plsc.store_scatter(pend_p, [pos], cv * 512 + u, mask=m)
            bc = jnp.take_along_axis(pc, lane15, axis=0,
                                     mode="promise_in_bounds")
            return npv + bc
        npv = lax.fori_loop(0, SCAN_C // L, scan_vec, npv0)
        return npv[0]

    def do_chunk(k, np_, rb, cb, rb_next, cb_next):
        wait_stage(k, rb, cb)
        np_ = scan_chunk(k, np_, rb, cb)

        @pl.when(k + 1 < NCHUNK)
        def _():
            start_stage(k + 1, rb_next, cb_next)

        nfull = jnp.where(np_ >= BMIN * GB, np_ // GB, 0)
        process_batches(nfull)
        rem = np_ - nfull * GB

        @pl.when(nfull > 0)
        def _():
            for t in range(GB // L):
                v = pend_p[pl.ds(nfull * GB + t * L, L)]
                pend_p[pl.ds(t * L, L)] = v
        return rem

    start_stage(0, rowb0, colb0)

    def pairchunk(p, np_):
        k0 = p * 2
        np_ = do_chunk(k0, np_, rowb0, colb0, rowb1, colb1)
        np_ = do_chunk(k0 + 1, np_, rowb1, colb1, rowb0, colb0)
        return np_
    np_ = lax.fori_loop(0, NCHUNK // 2, pairchunk, jnp.int32(0))

    for t in range(GB // L):
        pend_p[pl.ds(np_ + t * L, L)] = jnp.full((L,), OWN, jnp.int32)
    process_batches((np_ + GB - 1) // GB)

    @pl.when(g < NW - 1)
    def _():
        pltpu.sync_copy(acc.at[pl.ds(0, OWN)], out_hbm.at[pl.ds(base, OWN)])

    @pl.when(g == NW - 1)
    def _():
        pltpu.sync_copy(acc.at[pl.ds(0, N - (NW - 1) * OWN)],
                        out_hbm.at[pl.ds(base, N - (NW - 1) * OWN)])


_segsum = functools.partial(
    pl.kernel,
    out_type=jax.ShapeDtypeStruct((N, D), jnp.float32),
    mesh=plsc.VectorSubcoreMesh(
        core_axis_name="c", subcore_axis_name="s", num_cores=NC, num_subcores=NS
    ),
    scratch_types=[
        pltpu.VMEM((SCAN_C,), jnp.int32),
        pltpu.VMEM((SCAN_C,), jnp.int32),
        pltpu.VMEM((SCAN_C,), jnp.int32),
        pltpu.VMEM((SCAN_C,), jnp.int32),
        pltpu.VMEM((PEND,), jnp.int32),
        pltpu.VMEM((GB // 2,), jnp.int32),
        pltpu.VMEM((GB // 2,), jnp.int32),
        pltpu.VMEM((GB // 2,), jnp.int32),
        pltpu.VMEM((GB // 2,), jnp.int32),
        pltpu.VMEM((GB, D), jnp.float32),
        pltpu.VMEM((GB, D), jnp.float32),
        pltpu.VMEM((ACC_ROWS, D), jnp.float32),
        pltpu.SemaphoreType.DMA,
        pltpu.SemaphoreType.DMA,
        pltpu.SemaphoreType.DMA,
        pltpu.SemaphoreType.DMA,
        pltpu.SemaphoreType.DMA,
        pltpu.SemaphoreType.DMA,
    ],
    compiler_params=pltpu.CompilerParams(needs_layout_passes=False),
)(_segsum_body)


BLK = 1000
NBLK = N // BLK


def _mlp_body(x_ref, agg_ref, w1_ref, b1_ref, w2_ref, b2_ref,
              h_ref, s1_ref, s2_ref):
    i = pl.program_id(0)
    xa = x_ref[...] + agg_ref[...]
    h1 = jnp.maximum(
        jnp.dot(xa, w1_ref[...], preferred_element_type=jnp.float32) + b1_ref[...], 0.0)
    h = jnp.maximum(
        jnp.dot(h1, w2_ref[...], preferred_element_type=jnp.float32) + b2_ref[...], 0.0)
    h_ref[...] = h

    @pl.when(i == 0)
    def _():
        s1_ref[...] = jnp.zeros_like(s1_ref)
        s2_ref[...] = jnp.zeros_like(s2_ref)

    s1_ref[pl.ds(i, 1), :] = jnp.sum(h, axis=0, keepdims=True)
    s2_ref[pl.ds(i, 1), :] = jnp.sum(h * h, axis=0, keepdims=True)


def _mlp_stats(x, agg, W1, b1, W2, b2):
    return pl.pallas_call(
        _mlp_body,
        grid=(NBLK,),
        in_specs=[
            pl.BlockSpec((BLK, D), lambda i: (i, 0)),
            pl.BlockSpec((BLK, D), lambda i: (i, 0)),
            pl.BlockSpec((D, 2 * D), lambda i: (0, 0)),
            pl.BlockSpec((2 * D,), lambda i: (0,)),
            pl.BlockSpec((2 * D, D), lambda i: (0, 0)),
            pl.BlockSpec((D,), lambda i: (0,)),
        ],
        out_specs=[
            pl.BlockSpec((BLK, D), lambda i: (i, 0)),
            pl.BlockSpec((16, D), lambda i: (0, 0)),
            pl.BlockSpec((16, D), lambda i: (0, 0)),
        ],
        out_shape=[
            jax.ShapeDtypeStruct((N, D), jnp.float32),
            jax.ShapeDtypeStruct((16, D), jnp.float32),
            jax.ShapeDtypeStruct((16, D), jnp.float32),
        ],
    )(x, agg, W1, b1, W2, b2)


def _norm_body(h_ref, agg_ref, sc_ref, bi_ref, o_ref):
    m = (agg_ref[...] > 0).astype(jnp.float32)
    o_ref[...] = (h_ref[...] * sc_ref[0:1, :] + bi_ref[0:1, :]) * m


def _norm_mask(h, agg, scale, bias):
    return pl.pallas_call(
        _norm_body,
        grid=(NBLK,),
        in_specs=[
            pl.BlockSpec((BLK, D), lambda i: (i, 0)),
            pl.BlockSpec((BLK, D), lambda i: (i, 0)),
            pl.BlockSpec((8, D), lambda i: (0, 0)),
            pl.BlockSpec((8, D), lambda i: (0, 0)),
        ],
        out_specs=pl.BlockSpec((BLK, D), lambda i: (i, 0)),
        out_shape=jax.ShapeDtypeStruct((N, D), jnp.float32),
    )(h, agg, scale, bias)


def kernel(x, edge_index, W1, b1, W2, b2, gamma, beta):
    row = edge_index[0]
    col = edge_index[1]
    agg = _segsum(x, row, col)
    h, s1, s2 = _mlp_stats(x, agg, W1, b1, W2, b2)
    mean = jnp.sum(s1, axis=0) / N
    var = jnp.sum(s2, axis=0) / N - mean * mean
    rstd = 1.0 / jnp.sqrt(var + BN_EPS)
    inv_sqrt_n = 1.0 / jnp.sqrt(jnp.float32(N))
    scale = gamma * rstd * inv_sqrt_n
    bias = (beta - mean * gamma * rstd) * inv_sqrt_n
    scale_b = jnp.broadcast_to(scale[None, :], (8, D))
    bias_b = jnp.broadcast_to(bias[None, :], (8, D))
    return _norm_mask(h, agg, scale_b, bias_b)


# speedup vs baseline: 1.0602x; 1.0186x over previous
import functools

import jax
import jax.numpy as jnp
from jax import lax
from jax.experimental import pallas as pl
from jax.experimental.pallas import tpu as pltpu
from jax.experimental.pallas import tpu_sc as plsc

N = 10000
E = 160000
D = 256
BN_EPS = 1e-05

NC = 2
NS = 16
L = 16
NW = NC * NS

OWN = 320
ACC_ROWS = OWN + 8
SCAN_C = 1600
NCHUNK = E // SCAN_C
GB = 64
BMIN = 4
PEND = SCAN_C + (BMIN + 1) * GB


def _segsum_body(x_hbm, row_hbm, col_hbm, out_hbm,
                 rowb0, colb0, rowb1, colb1, pend_p,
                 gidx0a, gidx0b, gidx1a, gidx1b, grows0, grows1, acc,
                 semr, semc, sg0a, sg0b, sg1a, sg1b):
    c = lax.axis_index("c")
    s = lax.axis_index("s")
    g = c * NS + s
    base = g * OWN
    lane15 = jnp.full((L,), L - 1, jnp.int32)

    def zero_row(r, _):
        for j in range(D // L):
            acc[r, pl.ds(j * L, L)] = jnp.zeros((L,), jnp.float32)
        return ()
    lax.fori_loop(0, ACC_ROWS, zero_row, ())

    def start_stage(k, rb, cb):
        pltpu.async_copy(row_hbm.at[pl.ds(k * SCAN_C, SCAN_C)], rb, semr)
        pltpu.async_copy(col_hbm.at[pl.ds(k * SCAN_C, SCAN_C)], cb, semc)

    def wait_stage(k, rb, cb):
        pltpu.make_async_copy(row_hbm.at[pl.ds(k * SCAN_C, SCAN_C)],
                              rb, semr).wait()
        pltpu.make_async_copy(col_hbm.at[pl.ds(k * SCAN_C, SCAN_C)],
                              cb, semc).wait()

    HB = GB // 2

    def start_gather(b, slot):
        boff = b * GB
        gia, gib = (gidx0a, gidx0b) if slot == 0 else (gidx1a, gidx1b)
        gr = grows0 if slot == 0 else grows1
        sema, semb = (sg0a, sg0b) if slot == 0 else (sg1a, sg1b)
        for t in range(HB // L):
            gia[pl.ds(t * L, L)] = jnp.right_shift(
                pend_p[pl.ds(boff + t * L, L)], 9)
        for t in range(HB // L):
            gib[pl.ds(t * L, L)] = jnp.right_shift(
                pend_p[pl.ds(boff + HB + t * L, L)], 9)
        pltpu.async_copy(x_hbm.at[gia], gr.at[pl.ds(0, HB)], sema)
        pltpu.async_copy(x_hbm.at[gib], gr.at[pl.ds(HB, HB)], semb)

    def wait_gather(slot):
        gia, gib = (gidx0a, gidx0b) if slot == 0 else (gidx1a, gidx1b)
        gr = grows0 if slot == 0 else grows1
        sema, semb = (sg0a, sg0b) if slot == 0 else (sg1a, sg1b)
        pltpu.make_async_copy(x_hbm.at[gia], gr.at[pl.ds(0, HB)], sema).wait()
        pltpu.make_async_copy(x_hbm.at[gib], gr.at[pl.ds(HB, HB)], semb).wait()

    def add_batch(b, slot):
        boff = b * GB
        gbuf = grows0 if slot == 0 else grows1

        def grp(t, _):
            dv = pend_p[pl.ds(boff + t * L, L)] & 511
            for e in range(L):
                d = dv[e]
                gr = t * L + e
                for j in range(D // L):
                    plsc.addupdate(acc.at[d, pl.ds(j * L, L)],
                                   gbuf[gr, pl.ds(j * L, L)])
            return ()
        lax.fori_loop(0, GB // L, grp, ())

    def process_batches(nfull):
        @pl.when(nfull > 0)
        def _():
            start_gather(0, 0)

            def pair(p, _):
                b0 = p * 2

                @pl.when(b0 + 1 < nfull)
                def _():
                    start_gather(b0 + 1, 1)
                wait_gather(0)
                add_batch(b0, 0)

                @pl.when(b0 + 1 < nfull)
                def _():
                    @pl.when(b0 + 2 < nfull)
                    def _():
                        start_gather(b0 + 2, 0)
                    wait_gather(1)
                    add_batch(b0 + 1, 1)
                return ()
            lax.fori_loop(0, (nfull + 1) // 2, pair, ())

    def scan_chunk(k, np_, rb, cb):
        npv0 = jnp.full((L,), np_, jnp.int32)

        def scan_vec(i, npv):
            r = rb[pl.ds(i * L, L)]
            cv = cb[pl.ds(i * L, L)]
            u = r - base
            m = (u >= 0) & (u < OWN)
            pc = plsc.cumsum(m.astype(jnp.int32))
            pos = npv + pc - 1
            plsc.store_scatter(pend_p, [pos], cv * 512 + u, mask=m)
            bc = jnp.take_along_axis(pc, lane15, axis=0,
                                     mode="promise_in_bounds")
            return npv + bc
        npv = lax.fori_loop(0, SCAN_C // L, scan_vec, npv0, unroll=4)
        return npv[0]

    def do_chunk(k, np_, rb, cb, rb_next, cb_next):
        wait_stage(k, rb, cb)
        np_ = scan_chunk(k, np_, rb, cb)

        @pl.when(k + 1 < NCHUNK)
        def _():
            start_stage(k + 1, rb_next, cb_next)

        nfull = jnp.where(np_ >= BMIN * GB, np_ // GB, 0)
        process_batches(nfull)
        rem = np_ - nfull * GB

        @pl.when(nfull > 0)
        def _():
            for t in range(GB // L):
                v = pend_p[pl.ds(nfull * GB + t * L, L)]
                pend_p[pl.ds(t * L, L)] = v
        return rem

    start_stage(0, rowb0, colb0)

    def pairchunk(p, np_):
        k0 = p * 2
        np_ = do_chunk(k0, np_, rowb0, colb0, rowb1, colb1)
        np_ = do_chunk(k0 + 1, np_, rowb1, colb1, rowb0, colb0)
        return np_
    np_ = lax.fori_loop(0, NCHUNK // 2, pairchunk, jnp.int32(0))

    for t in range(GB // L):
        pend_p[pl.ds(np_ + t * L, L)] = jnp.full((L,), OWN, jnp.int32)
    process_batches((np_ + GB - 1) // GB)

    @pl.when(g < NW - 1)
    def _():
        pltpu.sync_copy(acc.at[pl.ds(0, OWN)], out_hbm.at[pl.ds(base, OWN)])

    @pl.when(g == NW - 1)
    def _():
        pltpu.sync_copy(acc.at[pl.ds(0, N - (NW - 1) * OWN)],
                        out_hbm.at[pl.ds(base, N - (NW - 1) * OWN)])


_segsum = functools.partial(
    pl.kernel,
    out_type=jax.ShapeDtypeStruct((N, D), jnp.float32),
    mesh=plsc.VectorSubcoreMesh(
        core_axis_name="c", subcore_axis_name="s", num_cores=NC, num_subcores=NS
    ),
    scratch_types=[
        pltpu.VMEM((SCAN_C,), jnp.int32),
        pltpu.VMEM((SCAN_C,), jnp.int32),
        pltpu.VMEM((SCAN_C,), jnp.int32),
        pltpu.VMEM((SCAN_C,), jnp.int32),
        pltpu.VMEM((PEND,), jnp.int32),
        pltpu.VMEM((GB // 2,), jnp.int32),
        pltpu.VMEM((GB // 2,), jnp.int32),
        pltpu.VMEM((GB // 2,), jnp.int32),
        pltpu.VMEM((GB // 2,), jnp.int32),
        pltpu.VMEM((GB, D), jnp.float32),
        pltpu.VMEM((GB, D), jnp.float32),
        pltpu.VMEM((ACC_ROWS, D), jnp.float32),
        pltpu.SemaphoreType.DMA,
        pltpu.SemaphoreType.DMA,
        pltpu.SemaphoreType.DMA,
        pltpu.SemaphoreType.DMA,
        pltpu.SemaphoreType.DMA,
        pltpu.SemaphoreType.DMA,
    ],
    compiler_params=pltpu.CompilerParams(needs_layout_passes=False),
)(_segsum_body)


BLK = 1000
NBLK = N // BLK


def _mlp_body(x_ref, agg_ref, w1_ref, b1_ref, w2_ref, b2_ref,
              h_ref, s1_ref, s2_ref):
    i = pl.program_id(0)
    xa = x_ref[...] + agg_ref[...]
    h1 = jnp.maximum(
        jnp.dot(xa, w1_ref[...], preferred_element_type=jnp.float32) + b1_ref[...], 0.0)
    h = jnp.maximum(
        jnp.dot(h1, w2_ref[...], preferred_element_type=jnp.float32) + b2_ref[...], 0.0)
    h_ref[...] = h

    @pl.when(i == 0)
    def _():
        s1_ref[...] = jnp.zeros_like(s1_ref)
        s2_ref[...] = jnp.zeros_like(s2_ref)

    s1_ref[pl.ds(i, 1), :] = jnp.sum(h, axis=0, keepdims=True)
    s2_ref[pl.ds(i, 1), :] = jnp.sum(h * h, axis=0, keepdims=True)


def _mlp_stats(x, agg, W1, b1, W2, b2):
    return pl.pallas_call(
        _mlp_body,
        grid=(NBLK,),
        in_specs=[
            pl.BlockSpec((BLK, D), lambda i: (i, 0)),
            pl.BlockSpec((BLK, D), lambda i: (i, 0)),
            pl.BlockSpec((D, 2 * D), lambda i: (0, 0)),
            pl.BlockSpec((2 * D,), lambda i: (0,)),
            pl.BlockSpec((2 * D, D), lambda i: (0, 0)),
            pl.BlockSpec((D,), lambda i: (0,)),
        ],
        out_specs=[
            pl.BlockSpec((BLK, D), lambda i: (i, 0)),
            pl.BlockSpec((16, D), lambda i: (0, 0)),
            pl.BlockSpec((16, D), lambda i: (0, 0)),
        ],
        out_shape=[
            jax.ShapeDtypeStruct((N, D), jnp.float32),
            jax.ShapeDtypeStruct((16, D), jnp.float32),
            jax.ShapeDtypeStruct((16, D), jnp.float32),
        ],
    )(x, agg, W1, b1, W2, b2)


def _norm_body(h_ref, agg_ref, sc_ref, bi_ref, o_ref):
    m = (agg_ref[...] > 0).astype(jnp.float32)
    o_ref[...] = (h_ref[...] * sc_ref[0:1, :] + bi_ref[0:1, :]) * m


def _norm_mask(h, agg, scale, bias):
    return pl.pallas_call(
        _norm_body,
        grid=(NBLK,),
        in_specs=[
            pl.BlockSpec((BLK, D), lambda i: (i, 0)),
            pl.BlockSpec((BLK, D), lambda i: (i, 0)),
            pl.BlockSpec((8, D), lambda i: (0, 0)),
            pl.BlockSpec((8, D), lambda i: (0, 0)),
        ],
        out_specs=pl.BlockSpec((BLK, D), lambda i: (i, 0)),
        out_shape=jax.ShapeDtypeStruct((N, D), jnp.float32),
    )(h, agg, scale, bias)


def kernel(x, edge_index, W1, b1, W2, b2, gamma, beta):
    row = edge_index[0]
    col = edge_index[1]
    agg = _segsum(x, row, col)
    h, s1, s2 = _mlp_stats(x, agg, W1, b1, W2, b2)
    mean = jnp.sum(s1, axis=0) / N
    var = jnp.sum(s2, axis=0) / N - mean * mean
    rstd = 1.0 / jnp.sqrt(var + BN_EPS)
    inv_sqrt_n = 1.0 / jnp.sqrt(jnp.float32(N))
    scale = gamma * rstd * inv_sqrt_n
    bias = (beta - mean * gamma * rstd) * inv_sqrt_n
    scale_b = jnp.broadcast_to(scale[None, :], (8, D))
    bias_b = jnp.broadcast_to(bias[None, :], (8, D))
    return _norm_mask(h, agg, scale_b, bias_b)
